# baseline (speedup 1.0000x reference)
import functools

import jax
import jax.numpy as jnp
from jax import lax
from jax.experimental import pallas as pl
from jax.experimental.pallas import tpu as pltpu
from jax.experimental.pallas import tpu_sc as plsc

_N = 10000
_E = 320000
_D = 128
_DE = 16
_H = 128
_NP = 10240
_NTILES = 32
_EPW = _E // _NTILES
_CH = 40
_NCH = _EPW // _CH
_RPT = _NP // 16
_BN = 1024


def _node_proj_body(x_ref, wq_ref, bq_ref, wk_ref, bk_ref, eq_ref, ek_ref):
    x = x_ref[...]
    dn = (((1,), (1,)), ((), ()))
    eq_ref[...] = lax.dot_general(x, wq_ref[...], dn,
                                  preferred_element_type=jnp.float32) + bq_ref[...]
    ek_ref[...] = lax.dot_general(x, wk_ref[...], dn,
                                  preferred_element_type=jnp.float32) + bk_ref[...]


def _node_proj(x, wq, bq2, wk, bk2):
    return pl.pallas_call(
        _node_proj_body,
        grid=(pl.cdiv(_N, _BN),),
        in_specs=[
            pl.BlockSpec((_BN, _D), lambda i: (i, 0)),
            pl.BlockSpec((_H, _D), lambda i: (0, 0)),
            pl.BlockSpec((1, _H), lambda i: (0, 0)),
            pl.BlockSpec((_H, _D), lambda i: (0, 0)),
            pl.BlockSpec((1, _H), lambda i: (0, 0)),
        ],
        out_specs=[
            pl.BlockSpec((_BN, _H), lambda i: (i, 0)),
            pl.BlockSpec((_BN, _H), lambda i: (i, 0)),
        ],
        out_shape=[
            jax.ShapeDtypeStruct((_N, _H), jnp.float32),
            jax.ShapeDtypeStruct((_N, _H), jnp.float32),
        ],
    )(x, wq, bq2, wk, bk2)


def _edge_proj_body(ef_ref, we_ref, be_ref, e_ref):
    e_ref[...] = lax.dot_general(
        ef_ref[...], we_ref[...], (((1,), (1,)), ((), ())),
        preferred_element_type=jnp.float32) + be_ref[...]


def _edge_proj(efeat, we, be2):
    be_blk = 1280
    return pl.pallas_call(
        _edge_proj_body,
        grid=(_E // be_blk,),
        in_specs=[
            pl.BlockSpec((be_blk, _DE), lambda i: (i, 0)),
            pl.BlockSpec((_H, _DE), lambda i: (0, 0)),
            pl.BlockSpec((1, _H), lambda i: (0, 0)),
        ],
        out_specs=pl.BlockSpec((be_blk, _H), lambda i: (i, 0)),
        out_shape=jax.ShapeDtypeStruct((_E, _H), jnp.float32),
    )(efeat, we, be2)


def _out_proj_body(a_ref, b_ref, wr_ref, br_ref, o_ref):
    acc = a_ref[...] + b_ref[...]
    o_ref[...] = lax.dot_general(
        acc, wr_ref[...], (((1,), (1,)), ((), ())),
        preferred_element_type=jnp.float32) + br_ref[...]


def _out_proj(ftp, wr, br2):
    nb = _NP // _BN
    return pl.pallas_call(
        _out_proj_body,
        grid=(nb,),
        in_specs=[
            pl.BlockSpec((_BN, _H), lambda i: (i, 0)),
            pl.BlockSpec((_BN, _H), lambda i, nb=nb: (i + nb, 0)),
            pl.BlockSpec((_H, _H), lambda i: (0, 0)),
            pl.BlockSpec((1, _H), lambda i: (0, 0)),
        ],
        out_specs=pl.BlockSpec((_BN, _H), lambda i: (i, 0)),
        out_shape=jax.ShapeDtypeStruct((_NP, _H), jnp.float32),
    )(ftp, ftp, wr, br2)


def _sc_edge_body(eq_hbm, ek_hbm, e_hbm, src_hbm, dst_hbm, out_hbm,
                  e0, q0, k0, m0, is0, id0, sd0,
                  e1, q1, k1, m1, is1, id1, sd1,
                  ft_sh,
                  sem_ix0, sem_ld0, sem_sc0, sem_ix1, sem_ld1, sem_sc1):
    c = lax.axis_index("c")
    s = lax.axis_index("s")
    wid = c * 16 + s
    sets = (
        dict(e=e0, q=q0, k=k0, m=m0, isrc=is0, idst=id0, sd=sd0,
             sem_ix=sem_ix0, sem_ld=sem_ld0, sem_sc=sem_sc0),
        dict(e=e1, q=q1, k=k1, m=m1, isrc=is1, idst=id1, sd=sd1,
             sem_ix=sem_ix1, sem_ld=sem_ld1, sem_sc=sem_sc1),
    )

    def _zero_row(r, carry):
        for j in range(8):
            m0[r, pl.ds(j * 16, 16)] = jnp.zeros((16,), jnp.float32)
        return carry

    lax.fori_loop(0, _CH, _zero_row, 0)
    for t in range(_RPT // _CH):
        pltpu.sync_copy(m0, ft_sh.at[pl.ds(s * _RPT + t * _CH, _CH)])
    plsc.subcore_barrier()

    def _issue_idx(ci, S):
        base = wid * _EPW + ci * _CH
        pltpu.async_copy(src_hbm.at[pl.ds(base, _CH)], S["isrc"], S["sem_ix"])
        pltpu.async_copy(dst_hbm.at[pl.ds(base, _CH)], S["idst"], S["sem_ix"])

    def _wait_idx(ci, S):
        base = wid * _EPW + ci * _CH
        pltpu.make_async_copy(src_hbm.at[pl.ds(base, _CH)], S["isrc"],
                              S["sem_ix"]).wait()
        pltpu.make_async_copy(dst_hbm.at[pl.ds(base, _CH)], S["idst"],
                              S["sem_ix"]).wait()

    def _issue_loads(ci, S):
        base = wid * _EPW + ci * _CH
        pltpu.async_copy(e_hbm.at[pl.ds(base, _CH)], S["e"], S["sem_ld"])
        pltpu.async_copy(eq_hbm.at[S["idst"]], S["q"], S["sem_ld"])
        pltpu.async_copy(ek_hbm.at[S["isrc"]], S["k"], S["sem_ld"])

    def _wait_loads(ci, S):
        base = wid * _EPW + ci * _CH
        pltpu.make_async_copy(e_hbm.at[pl.ds(base, _CH)], S["e"],
                              S["sem_ld"]).wait()
        pltpu.make_async_copy(eq_hbm.at[S["idst"]], S["q"], S["sem_ld"]).wait()
        pltpu.make_async_copy(ek_hbm.at[S["isrc"]], S["k"], S["sem_ld"]).wait()

    def _wait_scatter(S):
        pltpu.make_async_copy(S["m"], ft_sh.at[S["sd"]], S["sem_sc"]).wait()

    def _process(ci, i2, S, T):
        _wait_loads(ci, S)

        @pl.when(i2 >= 1)
        def _():
            _wait_scatter(S)

        for off in (0, 16, 24):
            S["sd"][pl.ds(off, 16)] = S["idst"][pl.ds(off, 16)]

        @pl.when(ci + 2 < _NCH)
        def _():
            _issue_idx(ci + 2, S)

        @pl.when(ci + 1 < _NCH)
        def _():
            _wait_idx(ci + 1, T)
            _issue_loads(ci + 1, T)

        def _row(r, rc):
            for j in range(8):
                sl = pl.ds(j * 16, 16)
                v = S["e"][r, sl] + S["q"][r, sl] + S["k"][r, sl]
                S["m"][r, sl] = jnp.maximum(v, 0.0)
            return rc

        lax.fori_loop(0, _CH, _row, 0, unroll=2)
        pltpu.async_copy(S["m"], ft_sh.at[S["sd"]], S["sem_sc"], add=True)

    _issue_idx(0, sets[0])
    _issue_idx(1, sets[1])
    _wait_idx(0, sets[0])
    _issue_loads(0, sets[0])

    def _pair(i2, carry):
        c0 = i2 * 2
        _process(c0, i2, sets[0], sets[1])
        _process(c0 + 1, i2, sets[1], sets[0])
        return carry

    lax.fori_loop(0, _NCH // 2, _pair, 0)
    _wait_scatter(sets[0])
    _wait_scatter(sets[1])
    plsc.subcore_barrier()

    pltpu.sync_copy(ft_sh.at[pl.ds(s * _RPT, _RPT)],
                    out_hbm.at[pl.ds(c * _NP + s * _RPT, _RPT)])


@functools.lru_cache(maxsize=1)
def _sc_edge_kernel():
    buf = lambda: pltpu.VMEM((_CH, _H), jnp.float32)
    idx = lambda: pltpu.VMEM((_CH,), jnp.int32)
    return functools.partial(
        pl.kernel,
        out_type=jax.ShapeDtypeStruct((2 * _NP, _H), jnp.float32),
        mesh=plsc.VectorSubcoreMesh(core_axis_name="c", subcore_axis_name="s",
                                    num_cores=2, num_subcores=16),
        scratch_types=[
            buf(), buf(), buf(), buf(), idx(), idx(), idx(),
            buf(), buf(), buf(), buf(), idx(), idx(), idx(),
            pltpu.VMEM_SHARED((_NP, _H), jnp.float32),
            pltpu.SemaphoreType.DMA,
            pltpu.SemaphoreType.DMA,
            pltpu.SemaphoreType.DMA,
            pltpu.SemaphoreType.DMA,
            pltpu.SemaphoreType.DMA,
            pltpu.SemaphoreType.DMA,
        ],
    )(_sc_edge_body)


def kernel(nfeat, edge_index, efeat, Wq, bq, Wk, bk, We, be, Wr, br):
    src = edge_index[0]
    dst = edge_index[1]
    eq, ek = _node_proj(nfeat, Wq, bq.reshape(1, _H), Wk, bk.reshape(1, _H))
    e = _edge_proj(efeat, We, be.reshape(1, _H))
    ftp = _sc_edge_kernel()(eq, ek, e, src, dst)
    rst = _out_proj(ftp, Wr, br.reshape(1, _H))
    return rst[:_N]


# speedup vs baseline: 1.3178x; 1.3178x over previous
import functools

import jax
import jax.numpy as jnp
from jax import lax
from jax.experimental import pallas as pl
from jax.experimental.pallas import tpu as pltpu
from jax.experimental.pallas import tpu_sc as plsc

_N = 10000
_E = 320000
_D = 128
_DE = 16
_H = 128
_NP = 10240
_NTILES = 32
_EPW = _E // _NTILES
_CH = 40
_NCH = _EPW // _CH
_RPT = _NP // 16
_BN = 1024


def _node_proj_body(x_ref, wq_ref, bq_ref, wk_ref, bk_ref, eq_ref, ek_ref):
    x = x_ref[...]
    dn = (((1,), (1,)), ((), ()))
    eq_ref[...] = lax.dot_general(x, wq_ref[...], dn,
                                  preferred_element_type=jnp.float32) + bq_ref[...]
    ek_ref[...] = lax.dot_general(x, wk_ref[...], dn,
                                  preferred_element_type=jnp.float32) + bk_ref[...]


def _node_proj(x, wq, bq2, wk, bk2):
    return pl.pallas_call(
        _node_proj_body,
        grid=(pl.cdiv(_N, _BN),),
        in_specs=[
            pl.BlockSpec((_BN, _D), lambda i: (i, 0)),
            pl.BlockSpec((_H, _D), lambda i: (0, 0)),
            pl.BlockSpec((1, _H), lambda i: (0, 0)),
            pl.BlockSpec((_H, _D), lambda i: (0, 0)),
            pl.BlockSpec((1, _H), lambda i: (0, 0)),
        ],
        out_specs=[
            pl.BlockSpec((_BN, _H), lambda i: (i, 0)),
            pl.BlockSpec((_BN, _H), lambda i: (i, 0)),
        ],
        out_shape=[
            jax.ShapeDtypeStruct((_N, _H), jnp.float32),
            jax.ShapeDtypeStruct((_N, _H), jnp.float32),
        ],
    )(x, wq, bq2, wk, bk2)


def _edge_proj_body(ef_ref, we_ref, be_ref, e_ref):
    e_ref[...] = lax.dot_general(
        ef_ref[...], we_ref[...], (((1,), (1,)), ((), ())),
        preferred_element_type=jnp.float32) + be_ref[...]


def _edge_proj(efeat, we, be2):
    be_blk = 1280
    return pl.pallas_call(
        _edge_proj_body,
        grid=(_E // be_blk,),
        in_specs=[
            pl.BlockSpec((be_blk, _DE), lambda i: (i, 0)),
            pl.BlockSpec((_H, _DE), lambda i: (0, 0)),
            pl.BlockSpec((1, _H), lambda i: (0, 0)),
        ],
        out_specs=pl.BlockSpec((be_blk, _H), lambda i: (i, 0)),
        out_shape=jax.ShapeDtypeStruct((_E, _H), jnp.float32),
    )(efeat, we, be2)


def _out_proj_body(a_ref, b_ref, wr_ref, br_ref, o_ref):
    acc = a_ref[...] + b_ref[...]
    o_ref[...] = lax.dot_general(
        acc, wr_ref[...], (((1,), (1,)), ((), ())),
        preferred_element_type=jnp.float32) + br_ref[...]


def _out_proj(ftp, wr, br2):
    nb = _NP // _BN
    return pl.pallas_call(
        _out_proj_body,
        grid=(nb,),
        in_specs=[
            pl.BlockSpec((_BN, _H), lambda i: (i, 0)),
            pl.BlockSpec((_BN, _H), lambda i, nb=nb: (i + nb, 0)),
            pl.BlockSpec((_H, _H), lambda i: (0, 0)),
            pl.BlockSpec((1, _H), lambda i: (0, 0)),
        ],
        out_specs=pl.BlockSpec((_BN, _H), lambda i: (i, 0)),
        out_shape=jax.ShapeDtypeStruct((_NP, _H), jnp.float32),
    )(ftp, ftp, wr, br2)


def _sc_edge_body(eq_hbm, ek_hbm, e_hbm, src_hbm, dst_hbm, out_hbm,
                  e0, q0, k0, m0, is0, id0, sd0,
                  e1, q1, k1, m1, is1, id1, sd1,
                  ft_sh,
                  sem_ix0, sem_ld0, sem_sc0, sem_ix1, sem_ld1, sem_sc1):
    c = lax.axis_index("c")
    s = lax.axis_index("s")
    wid = c * 16 + s
    sets = (
        dict(e=e0, q=q0, k=k0, m=m0, isrc=is0, idst=id0, sd=sd0,
             sem_ix=sem_ix0, sem_ld=sem_ld0, sem_sc=sem_sc0),
        dict(e=e1, q=q1, k=k1, m=m1, isrc=is1, idst=id1, sd=sd1,
             sem_ix=sem_ix1, sem_ld=sem_ld1, sem_sc=sem_sc1),
    )

    def _zero_row(r, carry):
        for j in range(8):
            m0[r, pl.ds(j * 16, 16)] = jnp.zeros((16,), jnp.float32)
        return carry

    lax.fori_loop(0, _CH, _zero_row, 0)
    for t in range(_RPT // _CH):
        pltpu.sync_copy(m0, ft_sh.at[pl.ds(s * _RPT + t * _CH, _CH)])
    plsc.subcore_barrier()

    def _issue_idx(ci, S):
        base = wid * _EPW + ci * _CH
        pltpu.async_copy(src_hbm.at[pl.ds(base, _CH)], S["isrc"], S["sem_ix"])
        pltpu.async_copy(dst_hbm.at[pl.ds(base, _CH)], S["idst"], S["sem_ix"])

    def _wait_idx(ci, S):
        base = wid * _EPW + ci * _CH
        pltpu.make_async_copy(src_hbm.at[pl.ds(base, _CH)], S["isrc"],
                              S["sem_ix"]).wait()
        pltpu.make_async_copy(dst_hbm.at[pl.ds(base, _CH)], S["idst"],
                              S["sem_ix"]).wait()

    def _issue_loads(ci, S):
        base = wid * _EPW + ci * _CH
        pltpu.async_copy(e_hbm.at[pl.ds(base, _CH)], S["e"], S["sem_ld"])
        pltpu.async_copy(eq_hbm.at[S["idst"]], S["q"], S["sem_ld"])
        pltpu.async_copy(ek_hbm.at[S["isrc"]], S["k"], S["sem_ld"])

    def _wait_loads(ci, S):
        base = wid * _EPW + ci * _CH
        pltpu.make_async_copy(e_hbm.at[pl.ds(base, _CH)], S["e"],
                              S["sem_ld"]).wait()
        pltpu.make_async_copy(eq_hbm.at[S["idst"]], S["q"], S["sem_ld"]).wait()
        pltpu.make_async_copy(ek_hbm.at[S["isrc"]], S["k"], S["sem_ld"]).wait()

    def _wait_scatter(S):
        pltpu.make_async_copy(S["m"], ft_sh.at[S["sd"]], S["sem_sc"]).wait()

    def _process(ci, i2, S, T):
        _wait_loads(ci, S)

        @pl.when(i2 >= 1)
        def _():
            _wait_scatter(S)

        for off in (0, 16, 24):
            S["sd"][pl.ds(off, 16)] = S["idst"][pl.ds(off, 16)]

        @pl.when(ci + 2 < _NCH)
        def _():
            _issue_idx(ci + 2, S)

        @pl.when(ci + 1 < _NCH)
        def _():
            _wait_idx(ci + 1, T)
            _issue_loads(ci + 1, T)

        @plsc.parallel_loop(0, _CH, step=1, unroll=4)
        def _row(r):
            for j in range(8):
                sl = pl.ds(j * 16, 16)
                v = S["e"][r, sl] + S["q"][r, sl] + S["k"][r, sl]
                S["m"][r, sl] = jnp.maximum(v, 0.0)
        pltpu.async_copy(S["m"], ft_sh.at[S["sd"]], S["sem_sc"], add=True)

    _issue_idx(0, sets[0])
    _issue_idx(1, sets[1])
    _wait_idx(0, sets[0])
    _issue_loads(0, sets[0])

    def _pair(i2, carry):
        c0 = i2 * 2
        _process(c0, i2, sets[0], sets[1])
        _process(c0 + 1, i2, sets[1], sets[0])
        return carry

    lax.fori_loop(0, _NCH // 2, _pair, 0)
    _wait_scatter(sets[0])
    _wait_scatter(sets[1])
    plsc.subcore_barrier()

    pltpu.sync_copy(ft_sh.at[pl.ds(s * _RPT, _RPT)],
                    out_hbm.at[pl.ds(c * _NP + s * _RPT, _RPT)])


@functools.lru_cache(maxsize=1)
def _sc_edge_kernel():
    buf = lambda: pltpu.VMEM((_CH, _H), jnp.float32)
    idx = lambda: pltpu.VMEM((_CH,), jnp.int32)
    return functools.partial(
        pl.kernel,
        out_type=jax.ShapeDtypeStruct((2 * _NP, _H), jnp.float32),
        mesh=plsc.VectorSubcoreMesh(core_axis_name="c", subcore_axis_name="s",
                                    num_cores=2, num_subcores=16),
        scratch_types=[
            buf(), buf(), buf(), buf(), idx(), idx(), idx(),
            buf(), buf(), buf(), buf(), idx(), idx(), idx(),
            pltpu.VMEM_SHARED((_NP, _H), jnp.float32),
            pltpu.SemaphoreType.DMA,
            pltpu.SemaphoreType.DMA,
            pltpu.SemaphoreType.DMA,
            pltpu.SemaphoreType.DMA,
            pltpu.SemaphoreType.DMA,
            pltpu.SemaphoreType.DMA,
        ],
    )(_sc_edge_body)


def kernel(nfeat, edge_index, efeat, Wq, bq, Wk, bk, We, be, Wr, br):
    src = edge_index[0]
    dst = edge_index[1]
    eq, ek = _node_proj(nfeat, Wq, bq.reshape(1, _H), Wk, bk.reshape(1, _H))
    e = _edge_proj(efeat, We, be.reshape(1, _H))
    ftp = _sc_edge_kernel()(eq, ek, e, src, dst)
    rst = _out_proj(ftp, Wr, br.reshape(1, _H))
    return rst[:_N]


# speedup vs baseline: 1.3557x; 1.0288x over previous
import functools

import jax
import jax.numpy as jnp
from jax import lax
from jax.experimental import pallas as pl
from jax.experimental.pallas import tpu as pltpu
from jax.experimental.pallas import tpu_sc as plsc

_N = 10000
_E = 320000
_D = 128
_DE = 16
_H = 128
_NTILES = 32
_CH = 64
_NCH = 156
_NCHT = _E // _CH
_RPT = 632
_NP = 16 * _RPT
_BN = 632


def _node_proj_body(x_ref, wq_ref, bq_ref, wk_ref, bk_ref, eq_ref, ek_ref):
    x = x_ref[...]
    dn = (((1,), (1,)), ((), ()))
    eq_ref[...] = lax.dot_general(x, wq_ref[...], dn,
                                  preferred_element_type=jnp.float32) + bq_ref[...]
    ek_ref[...] = lax.dot_general(x, wk_ref[...], dn,
                                  preferred_element_type=jnp.float32) + bk_ref[...]


def _node_proj(x, wq, bq2, wk, bk2):
    return pl.pallas_call(
        _node_proj_body,
        grid=(pl.cdiv(_N, _BN),),
        in_specs=[
            pl.BlockSpec((_BN, _D), lambda i: (i, 0)),
            pl.BlockSpec((_H, _D), lambda i: (0, 0)),
            pl.BlockSpec((1, _H), lambda i: (0, 0)),
            pl.BlockSpec((_H, _D), lambda i: (0, 0)),
            pl.BlockSpec((1, _H), lambda i: (0, 0)),
        ],
        out_specs=[
            pl.BlockSpec((_BN, _H), lambda i: (i, 0)),
            pl.BlockSpec((_BN, _H), lambda i: (i, 0)),
        ],
        out_shape=[
            jax.ShapeDtypeStruct((_N, _H), jnp.float32),
            jax.ShapeDtypeStruct((_N, _H), jnp.float32),
        ],
    )(x, wq, bq2, wk, bk2)


def _edge_proj_body(ef_ref, wb_ref, bb_ref, e_ref):
    e_ref[...] = lax.dot_general(
        ef_ref[...], wb_ref[...], (((1,), (0,)), ((), ())),
        preferred_element_type=jnp.float32) + bb_ref[...]


def _edge_proj(efr, wbig, bbig):
    blk = 400
    return pl.pallas_call(
        _edge_proj_body,
        grid=(_E // 8 // blk,),
        in_specs=[
            pl.BlockSpec((blk, 8 * _DE), lambda i: (i, 0)),
            pl.BlockSpec((8 * _DE, 8 * _H), lambda i: (0, 0)),
            pl.BlockSpec((1, 8 * _H), lambda i: (0, 0)),
        ],
        out_specs=pl.BlockSpec((blk, 8 * _H), lambda i: (i, 0)),
        out_shape=jax.ShapeDtypeStruct((_E // 8, 8 * _H), jnp.float32),
    )(efr, wbig, bbig)


def _out_proj_body(a_ref, b_ref, wr_ref, br_ref, o_ref):
    acc = a_ref[...] + b_ref[...]
    o_ref[...] = lax.dot_general(
        acc, wr_ref[...], (((1,), (1,)), ((), ())),
        preferred_element_type=jnp.float32) + br_ref[...]


def _out_proj(ftp, wr, br2):
    nb = _NP // _BN
    return pl.pallas_call(
        _out_proj_body,
        grid=(nb,),
        in_specs=[
            pl.BlockSpec((_BN, _H), lambda i: (i, 0)),
            pl.BlockSpec((_BN, _H), lambda i, nb=nb: (i + nb, 0)),
            pl.BlockSpec((_H, _H), lambda i: (0, 0)),
            pl.BlockSpec((1, _H), lambda i: (0, 0)),
        ],
        out_specs=pl.BlockSpec((_BN, _H), lambda i: (i, 0)),
        out_shape=jax.ShapeDtypeStruct((_NP, _H), jnp.float32),
    )(ftp, ftp, wr, br2)


def _sc_edge_body(eq_hbm, ek_hbm, e_hbm, src_hbm, dst_hbm, out_hbm,
                  e0, q0, k0, is0, id0, sd0,
                  e1, q1, k1, is1, id1, sd1,
                  ft_sh,
                  sem_ix0, sem_ld0, sem_sc0, sem_ix1, sem_ld1, sem_sc1):
    c = lax.axis_index("c")
    s = lax.axis_index("s")
    wid = c * 16 + s
    sets = (
        dict(e=e0, q=q0, k=k0, isrc=is0, idst=id0, sd=sd0,
             sem_ix=sem_ix0, sem_ld=sem_ld0, sem_sc=sem_sc0),
        dict(e=e1, q=q1, k=k1, isrc=is1, idst=id1, sd=sd1,
             sem_ix=sem_ix1, sem_ld=sem_ld1, sem_sc=sem_sc1),
    )

    def _zero_row(r, carry):
        for j in range(8):
            q0[r, pl.ds(j * 16, 16)] = jnp.zeros((16,), jnp.float32)
        return carry

    lax.fori_loop(0, _CH, _zero_row, 0)
    for t in range(9):
        pltpu.sync_copy(q0, ft_sh.at[pl.ds(s * _RPT + t * _CH, _CH)])
    pltpu.sync_copy(q0.at[pl.ds(0, _RPT - 9 * _CH)],
                    ft_sh.at[pl.ds(s * _RPT + 9 * _CH, _RPT - 9 * _CH)])
    plsc.subcore_barrier()

    def _issue_idx(g, S):
        pltpu.async_copy(src_hbm.at[pl.ds(g * _CH, _CH)], S["isrc"],
                         S["sem_ix"])
        pltpu.async_copy(dst_hbm.at[pl.ds(g * _CH, _CH)], S["idst"],
                         S["sem_ix"])

    def _wait_idx(g, S):
        pltpu.make_async_copy(src_hbm.at[pl.ds(g * _CH, _CH)], S["isrc"],
                              S["sem_ix"]).wait()
        pltpu.make_async_copy(dst_hbm.at[pl.ds(g * _CH, _CH)], S["idst"],
                              S["sem_ix"]).wait()

    def _issue_loads(g, S):
        pltpu.async_copy(e_hbm.at[pl.ds(g * 8, _CH // 8)], S["e"],
                         S["sem_ld"])
        pltpu.async_copy(eq_hbm.at[S["idst"]], S["q"], S["sem_ld"])
        pltpu.async_copy(ek_hbm.at[S["isrc"]], S["k"], S["sem_ld"])

    def _wait_loads(g, S):
        pltpu.make_async_copy(e_hbm.at[pl.ds(g * 8, _CH // 8)], S["e"],
                              S["sem_ld"]).wait()
        pltpu.make_async_copy(eq_hbm.at[S["idst"]], S["q"], S["sem_ld"]).wait()
        pltpu.make_async_copy(ek_hbm.at[S["isrc"]], S["k"], S["sem_ld"]).wait()

    def _wait_scatter(S):
        pltpu.make_async_copy(S["q"], ft_sh.at[S["sd"]], S["sem_sc"]).wait()

    def _compute(S):
        @plsc.parallel_loop(0, _CH // 8, step=1, unroll=1)
        def _row(rr):
            for jj in range(8):
                r = rr * 8 + jj
                for j in range(8):
                    sl = pl.ds(j * 16, 16)
                    v = (S["e"][rr, pl.ds(jj * _H + j * 16, 16)]
                         + S["q"][r, sl] + S["k"][r, sl])
                    S["q"][r, sl] = jnp.maximum(v, 0.0)

    def _copy_sd(S):
        for off in range(0, _CH, 16):
            S["sd"][pl.ds(off, 16)] = S["idst"][pl.ds(off, 16)]

    def _process(ci, S, T):
        g = wid * _NCH + ci
        _wait_loads(g, S)
        _copy_sd(S)

        @pl.when(ci + 2 < _NCH)
        def _():
            _issue_idx(g + 2, S)

        @pl.when(ci + 1 < _NCH)
        def _():
            _wait_idx(g + 1, T)

            @pl.when(ci >= 1)
            def _():
                _wait_scatter(T)

            _issue_loads(g + 1, T)

        _compute(S)
        pltpu.async_copy(S["q"], ft_sh.at[S["sd"]], S["sem_sc"], add=True)

    _issue_idx(wid * _NCH, sets[0])
    _issue_idx(wid * _NCH + 1, sets[1])
    _wait_idx(wid * _NCH, sets[0])
    _issue_loads(wid * _NCH, sets[0])

    def _pair(i2, carry):
        c0 = i2 * 2
        _process(c0, sets[0], sets[1])
        _process(c0 + 1, sets[1], sets[0])
        return carry

    lax.fori_loop(0, _NCH // 2, _pair, 0)
    _wait_scatter(sets[0])
    _wait_scatter(sets[1])

    @pl.when(wid < _NCHT - _NTILES * _NCH)
    def _():
        g = _NTILES * _NCH + wid
        S = sets[0]
        _issue_idx(g, S)
        _wait_idx(g, S)
        _issue_loads(g, S)
        _wait_loads(g, S)
        _copy_sd(S)
        _compute(S)
        pltpu.sync_copy(S["q"], ft_sh.at[S["sd"]], add=True)

    plsc.subcore_barrier()
    pltpu.sync_copy(ft_sh.at[pl.ds(s * _RPT, _RPT)],
                    out_hbm.at[pl.ds(c * _NP + s * _RPT, _RPT)])


@functools.lru_cache(maxsize=1)
def _sc_edge_kernel():
    buf = lambda: pltpu.VMEM((_CH, _H), jnp.float32)
    idx = lambda: pltpu.VMEM((_CH,), jnp.int32)
    ebuf = lambda: pltpu.VMEM((_CH // 8, 8 * _H), jnp.float32)
    return functools.partial(
        pl.kernel,
        out_type=jax.ShapeDtypeStruct((2 * _NP, _H), jnp.float32),
        mesh=plsc.VectorSubcoreMesh(core_axis_name="c", subcore_axis_name="s",
                                    num_cores=2, num_subcores=16),
        scratch_types=[
            ebuf(), buf(), buf(), idx(), idx(), idx(),
            ebuf(), buf(), buf(), idx(), idx(), idx(),
            pltpu.VMEM_SHARED((_NP, _H), jnp.float32),
            pltpu.SemaphoreType.DMA,
            pltpu.SemaphoreType.DMA,
            pltpu.SemaphoreType.DMA,
            pltpu.SemaphoreType.DMA,
            pltpu.SemaphoreType.DMA,
            pltpu.SemaphoreType.DMA,
        ],
    )(_sc_edge_body)


def kernel(nfeat, edge_index, efeat, Wq, bq, Wk, bk, We, be, Wr, br):
    src = edge_index[0]
    dst = edge_index[1]
    efr = efeat.reshape(_E // 8, 8 * _DE)
    wbig = jax.scipy.linalg.block_diag(*([We.T] * 8))
    bbig = jnp.tile(be, 8).reshape(1, 8 * _H)
    eq, ek = _node_proj(nfeat, Wq, bq.reshape(1, _H), Wk, bk.reshape(1, _H))
    e = _edge_proj(efr, wbig, bbig)
    ftp = _sc_edge_kernel()(eq, ek, e, src, dst)
    rst = _out_proj(ftp, Wr, br.reshape(1, _H))
    return rst[:_N]
